# baseline (device time: 70343 ns/iter reference)
import jax
import jax.numpy as jnp
from jax import lax
from jax.experimental import pallas as pl
from jax.experimental.pallas import tpu as pltpu

N_DEV = 4
N_TOK = 2048
D = 512
H = 1024
N_EXP = 32
E_LOCAL = N_EXP // N_DEV
CHUNK = N_TOK // N_DEV


def kernel(x, router_W, route_idx, expert_W):
    def body(x_ref, rw_ref, idx_ref, ew_ref, out_ref,
             gates_ref, ewb_ref, send_buf, recv_buf, send_sems, recv_sems):
        p = lax.axis_index("i")

        barrier_sem = pltpu.get_barrier_semaphore()
        for k in range(1, N_DEV):
            nbr = lax.rem(p + k, N_DEV)
            pl.semaphore_signal(
                barrier_sem, inc=1,
                device_id=(nbr,), device_id_type=pl.DeviceIdType.MESH,
            )
        pl.semaphore_wait(barrier_sem, N_DEV - 1)

        scores = jnp.dot(x_ref[:, :], rw_ref[:, :],
                         preferred_element_type=jnp.float32)
        s_max = jnp.max(scores, axis=-1, keepdims=True)
        probs = jnp.exp(scores - s_max)
        probs = probs / jnp.sum(probs, axis=-1, keepdims=True)
        eids = lax.broadcasted_iota(jnp.int32, (N_TOK, N_EXP), 1)
        idx0 = idx_ref[:, 0:1]
        idx1 = idx_ref[:, 1:2]
        p0 = jnp.sum(jnp.where(eids == idx0, probs, 0.0), axis=-1, keepdims=True)
        p1 = jnp.sum(jnp.where(eids == idx1, probs, 0.0), axis=-1, keepdims=True)
        denom = p0 + p1
        for j in range(E_LOCAL):
            e_j = p * E_LOCAL + j
            col = (jnp.where(idx0 == e_j, p0, 0.0)
                   + jnp.where(idx1 == e_j, p1, 0.0))
            gates_ref[:, j:j + 1] = col / denom

        for j in range(E_LOCAL):
            ewb_ref[j, :, :] = ew_ref[j, :, :].astype(jnp.bfloat16)

        def compute_chunk(c):
            r0 = c * CHUNK
            xc = x_ref[pl.ds(r0, CHUNK), :]
            gc = gates_ref[pl.ds(r0, CHUNK), :]
            acc = jnp.zeros((CHUNK, H), jnp.float32)
            for j in range(E_LOCAL):
                xg = (xc * gc[:, j:j + 1]).astype(jnp.bfloat16)
                acc = acc + jnp.dot(xg, ewb_ref[j, :, :],
                                    preferred_element_type=jnp.float32)
            return acc

        rdmas = []
        for k in range(N_DEV - 1):
            t = lax.rem(p + 1 + k, N_DEV)
            send_buf[k, :, :] = compute_chunk(t).astype(jnp.bfloat16)
            rdma = pltpu.make_async_remote_copy(
                src_ref=send_buf.at[k],
                dst_ref=recv_buf.at[N_DEV - 2 - k],
                send_sem=send_sems.at[k],
                recv_sem=recv_sems.at[N_DEV - 2 - k],
                device_id=(t,),
                device_id_type=pl.DeviceIdType.MESH,
            )
            rdma.start()
            rdmas.append(rdma)

        total = compute_chunk(p)
        for k in range(N_DEV - 1):
            rdmas[k].wait_recv()
            total = total + recv_buf[N_DEV - 2 - k, :, :].astype(jnp.float32)
        out_ref[:, :] = total
        for k in range(N_DEV - 1):
            rdmas[k].wait_send()

    return pl.pallas_call(
        body,
        out_shape=jax.ShapeDtypeStruct((CHUNK, H), jnp.float32),
        in_specs=[pl.BlockSpec(memory_space=pltpu.VMEM)] * 4,
        out_specs=pl.BlockSpec(memory_space=pltpu.VMEM),
        scratch_shapes=[
            pltpu.VMEM((N_TOK, E_LOCAL), jnp.float32),
            pltpu.VMEM((E_LOCAL, D, H), jnp.bfloat16),
            pltpu.VMEM((N_DEV - 1, CHUNK, H), jnp.bfloat16),
            pltpu.VMEM((N_DEV - 1, CHUNK, H), jnp.bfloat16),
            pltpu.SemaphoreType.DMA((N_DEV - 1,)),
            pltpu.SemaphoreType.DMA((N_DEV - 1,)),
        ],
        compiler_params=pltpu.CompilerParams(
            collective_id=0, vmem_limit_bytes=100 * 1024 * 1024,
        ),
    )(x, router_W, route_idx, expert_W)


# device time: 57623 ns/iter; 1.2207x vs baseline; 1.2207x over previous
import jax
import jax.numpy as jnp
from jax import lax
from jax.experimental import pallas as pl
from jax.experimental.pallas import tpu as pltpu

N_DEV = 4
N_TOK = 2048
D = 512
H = 1024
N_EXP = 32
E_LOCAL = N_EXP // N_DEV
CHUNK = N_TOK // N_DEV
HALF = CHUNK // 2


def kernel(x, router_W, route_idx, expert_W):
    def body(x_ref, rw_ref, idx_ref, ew_ref, out_ref,
             gates_ref, ewb_ref, send_buf, recv_buf, send_sems, recv_sems):
        p = lax.axis_index("i")

        barrier_sem = pltpu.get_barrier_semaphore()
        for k in range(1, N_DEV):
            nbr = lax.rem(p + k, N_DEV)
            pl.semaphore_signal(
                barrier_sem, inc=1,
                device_id=(nbr,), device_id_type=pl.DeviceIdType.MESH,
            )
        pl.semaphore_wait(barrier_sem, N_DEV - 1)

        scores = jnp.dot(x_ref[:, :], rw_ref[:, :],
                         preferred_element_type=jnp.float32)
        s_max = jnp.max(scores, axis=-1, keepdims=True)
        probs = jnp.exp(scores - s_max)
        probs = probs / jnp.sum(probs, axis=-1, keepdims=True)
        eids = lax.broadcasted_iota(jnp.int32, (N_TOK, N_EXP), 1)
        idx0 = idx_ref[:, 0:1]
        idx1 = idx_ref[:, 1:2]
        p0 = jnp.sum(jnp.where(eids == idx0, probs, 0.0), axis=-1, keepdims=True)
        p1 = jnp.sum(jnp.where(eids == idx1, probs, 0.0), axis=-1, keepdims=True)
        denom = p0 + p1
        for j in range(E_LOCAL):
            e_j = p * E_LOCAL + j
            col = (jnp.where(idx0 == e_j, p0, 0.0)
                   + jnp.where(idx1 == e_j, p1, 0.0))
            gates_ref[:, j:j + 1] = col / denom

        for j in range(E_LOCAL):
            ewb_ref[j, :, :] = ew_ref[j, :, :].astype(jnp.bfloat16)

        def compute_half(c, h):
            r0 = c * CHUNK + h * HALF
            xcb = x_ref[pl.ds(r0, HALF), :].astype(jnp.bfloat16)
            gc = gates_ref[pl.ds(r0, HALF), :]
            acc = jnp.zeros((HALF, H), jnp.float32)
            for j in range(E_LOCAL):
                y = jnp.dot(xcb, ewb_ref[j, :, :],
                            preferred_element_type=jnp.float32)
                acc = acc + y * gc[:, j:j + 1]
            return acc

        rdmas = []
        for k in range(N_DEV - 1):
            t = lax.rem(p + 1 + k, N_DEV)
            for h in range(2):
                slot = 2 * (N_DEV - 2 - k) + h
                send_buf[2 * k + h, :, :] = (
                    compute_half(t, h).astype(jnp.bfloat16))
                rdma = pltpu.make_async_remote_copy(
                    src_ref=send_buf.at[2 * k + h],
                    dst_ref=recv_buf.at[slot],
                    send_sem=send_sems.at[2 * k + h],
                    recv_sem=recv_sems.at[slot],
                    device_id=(t,),
                    device_id_type=pl.DeviceIdType.MESH,
                )
                rdma.start()
                rdmas.append(rdma)

        own = [compute_half(p, 0), compute_half(p, 1)]
        for k in range(N_DEV - 1):
            for h in range(2):
                slot = 2 * (N_DEV - 2 - k) + h
                rdmas[2 * k + h].wait_recv()
                own[h] = own[h] + recv_buf[slot, :, :].astype(jnp.float32)
        out_ref[pl.ds(0, HALF), :] = own[0]
        out_ref[pl.ds(HALF, HALF), :] = own[1]
        for r in rdmas:
            r.wait_send()

    return pl.pallas_call(
        body,
        out_shape=jax.ShapeDtypeStruct((CHUNK, H), jnp.float32),
        in_specs=[pl.BlockSpec(memory_space=pltpu.VMEM)] * 4,
        out_specs=pl.BlockSpec(memory_space=pltpu.VMEM),
        scratch_shapes=[
            pltpu.VMEM((N_TOK, E_LOCAL), jnp.float32),
            pltpu.VMEM((E_LOCAL, D, H), jnp.bfloat16),
            pltpu.VMEM((2 * (N_DEV - 1), HALF, H), jnp.bfloat16),
            pltpu.VMEM((2 * (N_DEV - 1), HALF, H), jnp.bfloat16),
            pltpu.SemaphoreType.DMA((2 * (N_DEV - 1),)),
            pltpu.SemaphoreType.DMA((2 * (N_DEV - 1),)),
        ],
        compiler_params=pltpu.CompilerParams(
            collective_id=0, vmem_limit_bytes=100 * 1024 * 1024,
        ),
    )(x, router_W, route_idx, expert_W)
